# Initial kernel scaffold; baseline (speedup 1.0000x reference)
import jax
import jax.numpy as jnp
from jax.experimental import pallas as pl


def kernel(F, x):
    raise NotImplementedError("write your pallas kernel here")


# speedup vs baseline: 2.2394x; 2.2394x over previous
import functools

import jax
import jax.numpy as jnp
from jax import lax
from jax.experimental import pallas as pl
from jax.experimental.pallas import tpu as pltpu
from jax.experimental.pallas import tpu_sc as plsc

I_, K_, B_, D_ = 8, 4096, 1024, 128
KB = 1024
NKB = K_ // KB
_PREC = lax.Precision.DEFAULT


def _argmin_body(x_ref, F_ref, out_ref):
    i = pl.program_id(0)
    x = x_ref[0]
    x2 = jnp.sum(x * x, axis=1, keepdims=True)

    def chunk(c, carry):
        mval, midx = carry
        fc = F_ref[0, pl.ds(c * KB, KB), :]
        f2c = jnp.sum(fc * fc, axis=1)[None, :]
        xf = lax.dot_general(x, fc, (((1,), (1,)), ((), ())),
                             precision=_PREC,
                             preferred_element_type=jnp.float32)
        dist = (x2 + f2c) - 2.0 * xf
        cmin = jnp.min(dist, axis=1, keepdims=True)
        iot = lax.broadcasted_iota(jnp.int32, (B_, KB), 1) + c * KB
        cidx = jnp.min(jnp.where(dist == cmin, iot, K_),
                       axis=1, keepdims=True)
        upd = cmin < mval
        return (jnp.where(upd, cmin, mval), jnp.where(upd, cidx, midx))

    init = (jnp.full((B_, 1), jnp.inf, jnp.float32),
            jnp.zeros((B_, 1), jnp.int32))
    _, midx = lax.fori_loop(0, NKB, chunk, init)
    out_ref[0, 0, :] = midx[:, 0] + i * K_


_argmin_call = pl.pallas_call(
    _argmin_body,
    grid=(I_,),
    in_specs=[
        pl.BlockSpec((1, B_, D_), lambda i: (i, 0, 0)),
        pl.BlockSpec((1, K_, D_), lambda i: (i, 0, 0)),
    ],
    out_specs=pl.BlockSpec((1, 1, B_), lambda i: (i, 0, 0)),
    out_shape=jax.ShapeDtypeStruct((I_, 1, B_), jnp.int32),
)

_NC, _NS = 2, 16
_NW = _NC * _NS
_BT = I_ * B_
_BPW = _BT // _NW


@functools.cache
def _sc_gather_fn():
    mesh = plsc.VectorSubcoreMesh(core_axis_name="c", subcore_axis_name="s")

    @functools.partial(
        pl.kernel,
        mesh=mesh,
        out_type=jax.ShapeDtypeStruct((_BT, D_), jnp.float32),
        scratch_types=[
            pltpu.VMEM((_BPW,), jnp.int32),
            pltpu.VMEM((_BPW, D_), jnp.float32),
            pltpu.SemaphoreType.DMA,
        ],
    )
    def _sc_gather(table_hbm, idx_hbm, out_hbm, idx_v, rows_v, sem):
        wid = lax.axis_index("s") * _NC + lax.axis_index("c")
        base = wid * _BPW
        pltpu.sync_copy(idx_hbm.at[pl.ds(base, _BPW)], idx_v)
        pltpu.async_copy(table_hbm.at[idx_v], rows_v, sem).wait()
        pltpu.sync_copy(rows_v, out_hbm.at[pl.ds(base, _BPW)])

    return _sc_gather


def kernel(F, x):
    idx3 = _argmin_call(x, F)
    idxf = idx3.reshape(_BT)
    out = _sc_gather_fn()(F.reshape(I_ * K_, D_), idxf)
    return out.reshape(I_, B_, D_)


# speedup vs baseline: 2.7811x; 1.2419x over previous
import functools

import jax
import jax.numpy as jnp
from jax import lax
from jax.experimental import pallas as pl
from jax.experimental.pallas import tpu as pltpu
from jax.experimental.pallas import tpu_sc as plsc

I_, K_, B_, D_ = 8, 4096, 1024, 128
KB = 1024
NKB = K_ // KB
_PREC = lax.Precision.DEFAULT


def _argmin_body(x_ref, F_ref, out_ref):
    i = pl.program_id(0)
    x = x_ref[0]
    xs = x * -2.0
    x2 = jnp.sum(x * x, axis=1, keepdims=True)
    Fm = F_ref[0]
    f2 = jnp.sum(Fm * Fm, axis=1)[None, :]
    xf2 = lax.dot_general(xs, Fm, (((1,), (1,)), ((), ())),
                          precision=_PREC,
                          preferred_element_type=jnp.float32)
    dist = (x2 + f2) + xf2

    rmin = dist[:, 0:128]
    for s in range(1, K_ // 128):
        rmin = jnp.minimum(rmin, dist[:, s * 128:(s + 1) * 128])
    cmin = jnp.min(rmin, axis=1, keepdims=True)

    big = jnp.float32(2.0 ** 30)
    iotaf = lax.broadcasted_iota(jnp.int32, (B_, 128), 1).astype(jnp.float32)
    acc = jnp.full((B_, 128), big, jnp.float32)
    for s in range(K_ // 128):
        cand = jnp.where(dist[:, s * 128:(s + 1) * 128] == cmin,
                         iotaf + jnp.float32(s * 128), big)
        acc = jnp.minimum(acc, cand)
    midx = jnp.min(acc, axis=1, keepdims=True).astype(jnp.int32)
    out_ref[0, 0, :] = midx[:, 0] + i * K_


_argmin_call = pl.pallas_call(
    _argmin_body,
    grid=(I_,),
    in_specs=[
        pl.BlockSpec((1, B_, D_), lambda i: (i, 0, 0)),
        pl.BlockSpec((1, K_, D_), lambda i: (i, 0, 0)),
    ],
    out_specs=pl.BlockSpec((1, 1, B_), lambda i: (i, 0, 0)),
    out_shape=jax.ShapeDtypeStruct((I_, 1, B_), jnp.int32),
)

_NC, _NS = 2, 16
_NW = _NC * _NS
_BT = I_ * B_
_BPW = _BT // _NW


@functools.cache
def _sc_gather_fn():
    mesh = plsc.VectorSubcoreMesh(core_axis_name="c", subcore_axis_name="s")

    @functools.partial(
        pl.kernel,
        mesh=mesh,
        out_type=jax.ShapeDtypeStruct((_BT, D_), jnp.float32),
        scratch_types=[
            pltpu.VMEM((_BPW,), jnp.int32),
            pltpu.VMEM((_BPW, D_), jnp.float32),
            pltpu.SemaphoreType.DMA,
        ],
    )
    def _sc_gather(table_hbm, idx_hbm, out_hbm, idx_v, rows_v, sem):
        wid = lax.axis_index("s") * _NC + lax.axis_index("c")
        base = wid * _BPW
        pltpu.sync_copy(idx_hbm.at[pl.ds(base, _BPW)], idx_v)
        pltpu.async_copy(table_hbm.at[idx_v], rows_v, sem).wait()
        pltpu.sync_copy(rows_v, out_hbm.at[pl.ds(base, _BPW)])

    return _sc_gather


def kernel(F, x):
    idx3 = _argmin_call(x, F)
    idxf = idx3.reshape(_BT)
    out = _sc_gather_fn()(F.reshape(I_ * K_, D_), idxf)
    return out.reshape(I_, B_, D_)


# speedup vs baseline: 2.8253x; 1.0159x over previous
import functools

import jax
import jax.numpy as jnp
from jax import lax
from jax.experimental import pallas as pl
from jax.experimental.pallas import tpu as pltpu
from jax.experimental.pallas import tpu_sc as plsc

I_, K_, B_, D_ = 8, 4096, 1024, 128
KB = 1024
NKB = K_ // KB
_PREC = lax.Precision.DEFAULT


def _argmin_body(x_ref, F_ref, out_ref):
    i = pl.program_id(0)
    x = x_ref[0]
    xs = x * -2.0
    x2 = jnp.sum(x * x, axis=1, keepdims=True)
    Fm = F_ref[0]
    f2 = jnp.sum(Fm * Fm, axis=1)[None, :]
    xf2 = lax.dot_general(xs, Fm, (((1,), (1,)), ((), ())),
                          precision=_PREC,
                          preferred_element_type=jnp.float32)
    iotaf = lax.broadcasted_iota(jnp.int32, (B_, 128), 1).astype(jnp.float32)
    acc_v = (x2 + f2[:, 0:128]) + xf2[:, 0:128]
    acc_i = iotaf
    for s in range(1, K_ // 128):
        d = (x2 + f2[:, s * 128:(s + 1) * 128]) + xf2[:, s * 128:(s + 1) * 128]
        lt = d < acc_v
        acc_v = jnp.where(lt, d, acc_v)
        acc_i = jnp.where(lt, iotaf + jnp.float32(s * 128), acc_i)

    big = jnp.float32(2.0 ** 30)
    gmin = jnp.min(acc_v, axis=1, keepdims=True)
    cand = jnp.where(acc_v == gmin, acc_i, big)
    midx = jnp.min(cand, axis=1, keepdims=True).astype(jnp.int32)
    out_ref[0, 0, :] = midx[:, 0] + i * K_


_argmin_call = pl.pallas_call(
    _argmin_body,
    grid=(I_,),
    in_specs=[
        pl.BlockSpec((1, B_, D_), lambda i: (i, 0, 0)),
        pl.BlockSpec((1, K_, D_), lambda i: (i, 0, 0)),
    ],
    out_specs=pl.BlockSpec((1, 1, B_), lambda i: (i, 0, 0)),
    out_shape=jax.ShapeDtypeStruct((I_, 1, B_), jnp.int32),
)

_NC, _NS = 2, 16
_NW = _NC * _NS
_BT = I_ * B_
_BPW = _BT // _NW


@functools.cache
def _sc_gather_fn():
    mesh = plsc.VectorSubcoreMesh(core_axis_name="c", subcore_axis_name="s")

    @functools.partial(
        pl.kernel,
        mesh=mesh,
        out_type=jax.ShapeDtypeStruct((_BT, D_), jnp.float32),
        scratch_types=[
            pltpu.VMEM((_BPW,), jnp.int32),
            pltpu.VMEM((_BPW, D_), jnp.float32),
            pltpu.SemaphoreType.DMA,
        ],
    )
    def _sc_gather(table_hbm, idx_hbm, out_hbm, idx_v, rows_v, sem):
        wid = lax.axis_index("s") * _NC + lax.axis_index("c")
        base = wid * _BPW
        pltpu.sync_copy(idx_hbm.at[pl.ds(base, _BPW)], idx_v)
        pltpu.async_copy(table_hbm.at[idx_v], rows_v, sem).wait()
        pltpu.sync_copy(rows_v, out_hbm.at[pl.ds(base, _BPW)])

    return _sc_gather


def kernel(F, x):
    idx3 = _argmin_call(x, F)
    idxf = idx3.reshape(_BT)
    out = _sc_gather_fn()(F.reshape(I_ * K_, D_), idxf)
    return out.reshape(I_, B_, D_)


# speedup vs baseline: 2.8261x; 1.0003x over previous
import functools

import jax
import jax.numpy as jnp
from jax import lax
from jax.experimental import pallas as pl
from jax.experimental.pallas import tpu as pltpu
from jax.experimental.pallas import tpu_sc as plsc

I_, K_, B_, D_ = 8, 4096, 1024, 128
KB = 1024
NKB = K_ // KB
_PREC = lax.Precision.DEFAULT


CH = 512
NCH = K_ // CH


def _argmin_body(x_ref, F_ref, out_ref):
    i = pl.program_id(0)
    x = x_ref[...]
    xs = x * -2.0
    x2 = jnp.sum(x * x, axis=1, keepdims=True)
    iotaf = lax.broadcasted_iota(jnp.int32, (B_, 128), 1).astype(jnp.float32)

    acc_v, acc_i = None, None
    for c in range(NCH):
        Fc = F_ref[c * CH:(c + 1) * CH, :]
        f2c = jnp.sum(Fc * Fc, axis=1)[None, :]
        xfc = lax.dot_general(xs, Fc, (((1,), (1,)), ((), ())),
                              precision=_PREC,
                              preferred_element_type=jnp.float32)
        for s in range(CH // 128):
            base = c * CH + s * 128
            d = (x2 + f2c[:, s * 128:(s + 1) * 128]) + xfc[:, s * 128:(s + 1) * 128]
            if acc_v is None:
                acc_v, acc_i = d, iotaf
            else:
                lt = d < acc_v
                acc_v = jnp.where(lt, d, acc_v)
                acc_i = jnp.where(lt, iotaf + jnp.float32(base), acc_i)

    big = jnp.float32(2.0 ** 30)
    gmin = jnp.min(acc_v, axis=1, keepdims=True)
    cand = jnp.where(acc_v == gmin, acc_i, big)
    midx = jnp.min(cand, axis=1, keepdims=True).astype(jnp.int32)
    out_ref[0, :] = midx[:, 0] + i * K_


_argmin_call = pl.pallas_call(
    _argmin_body,
    grid=(I_,),
    in_specs=[
        pl.BlockSpec((None, B_, D_), lambda i: (i, 0, 0)),
        pl.BlockSpec((None, K_, D_), lambda i: (i, 0, 0)),
    ],
    out_specs=pl.BlockSpec((None, 1, B_), lambda i: (i, 0, 0)),
    out_shape=jax.ShapeDtypeStruct((I_, 1, B_), jnp.int32),
)

_NC, _NS = 2, 16
_NW = _NC * _NS
_BT = I_ * B_
_BPW = _BT // _NW


@functools.cache
def _sc_gather_fn():
    mesh = plsc.VectorSubcoreMesh(core_axis_name="c", subcore_axis_name="s")

    @functools.partial(
        pl.kernel,
        mesh=mesh,
        out_type=jax.ShapeDtypeStruct((_BT, D_), jnp.float32),
        scratch_types=[
            pltpu.VMEM((_BPW,), jnp.int32),
            pltpu.VMEM((_BPW, D_), jnp.float32),
            pltpu.SemaphoreType.DMA,
        ],
    )
    def _sc_gather(table_hbm, idx_hbm, out_hbm, idx_v, rows_v, sem):
        wid = lax.axis_index("s") * _NC + lax.axis_index("c")
        base = wid * _BPW
        pltpu.sync_copy(idx_hbm.at[pl.ds(base, _BPW)], idx_v)
        pltpu.async_copy(table_hbm.at[idx_v], rows_v, sem).wait()
        pltpu.sync_copy(rows_v, out_hbm.at[pl.ds(base, _BPW)])

    return _sc_gather


def kernel(F, x):
    idx3 = _argmin_call(x, F)
    idxf = idx3.reshape(_BT)
    out = _sc_gather_fn()(F.reshape(I_ * K_, D_), idxf)
    return out.reshape(I_, B_, D_)
